# Initial kernel scaffold; baseline (speedup 1.0000x reference)
import jax
import jax.numpy as jnp
from jax.experimental import pallas as pl


def kernel(h, edge_index, W_fc, W_attn):
    raise NotImplementedError("write your pallas kernel here")


# speedup vs baseline: 15.1343x; 15.1343x over previous
import functools

import jax
import jax.numpy as jnp
from jax import lax
from jax.experimental import pallas as pl
from jax.experimental.pallas import tpu as pltpu
from jax.experimental.pallas import tpu_sc as plsc

N = 10000
E = 320000
D = 128

NC = 2
NS = 16
NW = NC * NS
EPT = E // NW
CH = 80
CHUNKS = EPT // CH
NP = 10240
RPT = NP // NS

BM = 400


def _tc_a(hc_ref, wfc_ref, ap_ref, z_ref, s_ref, m_ref):
    zb = jnp.dot(hc_ref[...], wfc_ref[...], preferred_element_type=jnp.float32)
    z_ref[...] = zb
    sb = jnp.dot(zb, ap_ref[...], preferred_element_type=jnp.float32)
    s_ref[...] = sb
    bmax = jnp.broadcast_to(jnp.max(sb, axis=0, keepdims=True), (8, 128))

    @pl.when(pl.program_id(0) == 0)
    def _():
        m_ref[...] = bmax

    @pl.when(pl.program_id(0) > 0)
    def _():
        m_ref[...] = jnp.maximum(m_ref[...], bmax)


def _dense_front(h_cat, W_fc, A_pad):
    return pl.pallas_call(
        _tc_a,
        grid=(N // BM,),
        in_specs=[
            pl.BlockSpec((BM, 3 * D), lambda i: (i, 0)),
            pl.BlockSpec((3 * D, D), lambda i: (0, 0)),
            pl.BlockSpec((D, 128), lambda i: (0, 0)),
        ],
        out_specs=[
            pl.BlockSpec((BM, D), lambda i: (i, 0)),
            pl.BlockSpec((BM, 128), lambda i: (i, 0)),
            pl.BlockSpec((8, 128), lambda i: (0, 0)),
        ],
        out_shape=[
            jax.ShapeDtypeStruct((N, D), jnp.float32),
            jax.ShapeDtypeStruct((N, 128), jnp.float32),
            jax.ShapeDtypeStruct((8, 128), jnp.float32),
        ],
    )(h_cat, W_fc, A_pad)


def _sc_edges(z, s1, s2, src, dst, m_splat):
    mesh = plsc.VectorSubcoreMesh(core_axis_name="c", subcore_axis_name="s")

    @functools.partial(
        pl.kernel,
        out_type=[
            jax.ShapeDtypeStruct((NC, NP, D), jnp.float32),
            jax.ShapeDtypeStruct((NW, 1, N), jnp.float32),
        ],
        mesh=mesh,
        compiler_params=pltpu.CompilerParams(
            needs_layout_passes=False, use_tc_tiling_on_sc=False),
        scratch_types=[
            pltpu.VMEM((N,), jnp.float32),
            pltpu.VMEM((N,), jnp.float32),
            pltpu.VMEM((1, N), jnp.float32),
            pltpu.VMEM((CH,), jnp.int32),
            pltpu.VMEM((1, CH), jnp.int32),
            pltpu.VMEM((CH, D), jnp.float32),
            pltpu.VMEM((16,), jnp.float32),
            pltpu.VMEM_SHARED((NP, D), jnp.float32),
            pltpu.SemaphoreType.DMA,
        ],
    )
    def k(z_hbm, s1_hbm, s2_hbm, src_hbm, dst2_hbm, m_hbm,
          acc_hbm, den_hbm,
          s1_v, s2_v, den_v, src_c, dst_c, rows_v, m_v,
          acc_sh, sem):
        cid = lax.axis_index("c")
        sid = lax.axis_index("s")
        wid = sid * NC + cid
        base = wid * EPT
        cbase = wid * CHUNKS

        pltpu.sync_copy(s1_hbm, s1_v)
        pltpu.sync_copy(s2_hbm, s2_v)
        pltpu.sync_copy(m_hbm, m_v)

        zf = jnp.zeros((16,), jnp.float32)

        def zero_den(i, carry):
            den_v[0, pl.ds(i * 16, 16)] = zf
            return carry

        lax.fori_loop(0, N // 16, zero_den, 0)

        def zero_rows(r, carry):
            for q in range(D // 16):
                rows_v[r, pl.ds(q * 16, 16)] = zf
            return carry

        lax.fori_loop(0, CH, zero_rows, 0)

        rbase = sid * RPT
        for t in range(RPT // CH):
            pltpu.sync_copy(rows_v, acc_sh.at[pl.ds(rbase + t * CH, CH)])

        plsc.subcore_barrier()

        mM = m_v[pl.ds(0, 16)]

        def chunk(c, carry):
            pltpu.sync_copy(src_hbm.at[pl.ds(base + c * CH, CH)], src_c)
            pltpu.sync_copy(dst2_hbm.at[pl.ds(cbase + c, 1)], dst_c)
            cp = pltpu.async_copy(z_hbm.at[src_c], rows_v, sem)
            pgs = []
            for g in range(CH // 16):
                si = src_c[pl.ds(g * 16, 16)]
                di = dst_c[0, pl.ds(g * 16, 16)]
                g1 = plsc.load_gather(s1_v, [si])
                g2 = plsc.load_gather(s2_v, [di])
                p = jnp.exp(g1 + g2 - mM)
                plsc.addupdate_scatter(den_v.at[0], [di], p)
                pgs.append(p)
            cp.wait()
            lanes = jnp.arange(16, dtype=jnp.int32)
            for g in range(CH // 16):
                pg = pgs[g]
                for k2 in range(16):
                    oh = (lanes == k2).astype(jnp.float32)
                    w = jnp.broadcast_to(jnp.sum(pg * oh), (16,))
                    r = g * 16 + k2
                    for q in range(D // 16):
                        rows_v[r, pl.ds(q * 16, 16)] = rows_v[r, pl.ds(q * 16, 16)] * w
            plsc.subcore_barrier()
            pltpu.sync_copy(rows_v, acc_sh.at[dst_c.at[0]], add=True)
            return carry

        lax.fori_loop(0, CHUNKS, chunk, 0)

        pltpu.sync_copy(den_v, den_hbm.at[wid])
        plsc.subcore_barrier()

        pltpu.sync_copy(acc_sh.at[pl.ds(rbase, RPT)],
                        acc_hbm.at[cid].at[pl.ds(rbase, RPT)])

    return k(z, s1, s2, src, dst, m_splat)


def _tc_c(acc_ref, dp_ref, o_ref):
    d = jnp.sum(dp_ref[:, 0, :], axis=0)
    inv = jnp.where(d > 0, 1.0 / d, 0.0)
    o_ref[...] = (acc_ref[0, :N] + acc_ref[1, :N]) * inv[:, None]


def _combine(acc, den_parts):
    return pl.pallas_call(
        _tc_c,
        out_shape=jax.ShapeDtypeStruct((N, D), jnp.float32),
    )(acc, den_parts)


def kernel(h, edge_index, W_fc, W_attn):
    h_pad = jnp.pad(h, ((1, 1), (0, 0)))
    h_cat = jnp.concatenate([h_pad[:-2], h, h_pad[2:]], axis=1)
    A_pad = jnp.pad(W_attn.reshape(2, D).T, ((0, 0), (0, 126)))

    z, S, m = _dense_front(h_cat, W_fc, A_pad)
    s1 = S[:, 0]
    s2 = S[:, 1]
    M = m[0, 0] + m[0, 1]
    m_splat = jnp.full((16,), M, jnp.float32)

    src = edge_index[0]
    dst = edge_index[1]
    dst2 = dst.reshape(E // CH, CH)
    acc, den_parts = _sc_edges(z, s1, s2, src, dst2, m_splat)
    return _combine(acc, den_parts)


# speedup vs baseline: 15.9779x; 1.0557x over previous
import functools

import jax
import jax.numpy as jnp
from jax import lax
from jax.experimental import pallas as pl
from jax.experimental.pallas import tpu as pltpu
from jax.experimental.pallas import tpu_sc as plsc

N = 10000
E = 320000
D = 128

NC = 2
NS = 16
NW = NC * NS
EPT = E // NW
CH = 80
CHUNKS = EPT // CH
NP = 10240
RPT = NP // NS

BM = 400


def _tc_a(hc_ref, wfc_ref, ap_ref, z_ref, s_ref, m_ref):
    zb = jnp.dot(hc_ref[...], wfc_ref[...], preferred_element_type=jnp.float32)
    z_ref[...] = zb
    sb = jnp.dot(zb, ap_ref[...], preferred_element_type=jnp.float32)
    s_ref[...] = sb
    bmax = jnp.broadcast_to(jnp.max(sb, axis=0, keepdims=True), (8, 128))

    @pl.when(pl.program_id(0) == 0)
    def _():
        m_ref[...] = bmax

    @pl.when(pl.program_id(0) > 0)
    def _():
        m_ref[...] = jnp.maximum(m_ref[...], bmax)


def _dense_front(h_cat, W_fc, A_pad):
    return pl.pallas_call(
        _tc_a,
        grid=(N // BM,),
        in_specs=[
            pl.BlockSpec((BM, 3 * D), lambda i: (i, 0)),
            pl.BlockSpec((3 * D, D), lambda i: (0, 0)),
            pl.BlockSpec((D, 128), lambda i: (0, 0)),
        ],
        out_specs=[
            pl.BlockSpec((BM, D), lambda i: (i, 0)),
            pl.BlockSpec((BM, 128), lambda i: (i, 0)),
            pl.BlockSpec((8, 128), lambda i: (0, 0)),
        ],
        out_shape=[
            jax.ShapeDtypeStruct((N, D), jnp.float32),
            jax.ShapeDtypeStruct((N, 128), jnp.float32),
            jax.ShapeDtypeStruct((8, 128), jnp.float32),
        ],
    )(h_cat, W_fc, A_pad)


def _sc_edges(z, s1, s2, src, dst, m_splat):
    mesh = plsc.VectorSubcoreMesh(core_axis_name="c", subcore_axis_name="s")

    @functools.partial(
        pl.kernel,
        out_type=[
            jax.ShapeDtypeStruct((NC, NP, D), jnp.float32),
            jax.ShapeDtypeStruct((NW, 1, N), jnp.float32),
        ],
        mesh=mesh,
        compiler_params=pltpu.CompilerParams(
            needs_layout_passes=False, use_tc_tiling_on_sc=False),
        scratch_types=[
            pltpu.VMEM((N,), jnp.float32),
            pltpu.VMEM((N,), jnp.float32),
            pltpu.VMEM((1, N), jnp.float32),
            pltpu.VMEM((CH,), jnp.int32),
            pltpu.VMEM((1, CH), jnp.int32),
            pltpu.VMEM((CH, D), jnp.float32),
            pltpu.VMEM((16,), jnp.float32),
            pltpu.VMEM_SHARED((NP, D), jnp.float32),
            pltpu.SemaphoreType.DMA,
        ],
    )
    def k(z_hbm, s1_hbm, s2_hbm, src_hbm, dst2_hbm, m_hbm,
          acc_hbm, den_hbm,
          s1_v, s2_v, den_v, src_c, dst_c, rows_v, m_v,
          acc_sh, sem):
        cid = lax.axis_index("c")
        sid = lax.axis_index("s")
        wid = sid * NC + cid
        base = wid * EPT
        cbase = wid * CHUNKS

        pltpu.sync_copy(s1_hbm, s1_v)
        pltpu.sync_copy(s2_hbm, s2_v)
        pltpu.sync_copy(m_hbm, m_v)

        zf = jnp.zeros((16,), jnp.float32)

        def zero_den(i, carry):
            den_v[0, pl.ds(i * 16, 16)] = zf
            return carry

        lax.fori_loop(0, N // 16, zero_den, 0)

        def zero_rows(r, carry):
            for q in range(D // 16):
                rows_v[r, pl.ds(q * 16, 16)] = zf
            return carry

        lax.fori_loop(0, CH, zero_rows, 0)

        rbase = sid * RPT
        for t in range(RPT // CH):
            pltpu.sync_copy(rows_v, acc_sh.at[pl.ds(rbase + t * CH, CH)])

        plsc.subcore_barrier()

        mM = m_v[pl.ds(0, 16)]

        def chunk(c, carry):
            pltpu.sync_copy(src_hbm.at[pl.ds(base + c * CH, CH)], src_c)
            pltpu.sync_copy(dst2_hbm.at[pl.ds(cbase + c, 1)], dst_c)
            cp = pltpu.async_copy(z_hbm.at[src_c], rows_v, sem)
            pgs = []
            for g in range(CH // 16):
                si = src_c[pl.ds(g * 16, 16)]
                di = dst_c[0, pl.ds(g * 16, 16)]
                g1 = plsc.load_gather(s1_v, [si])
                g2 = plsc.load_gather(s2_v, [di])
                p = jnp.exp(g1 + g2 - mM)
                plsc.addupdate_scatter(den_v.at[0], [di], p)
                pgs.append(p)
            cp.wait()
            lanes = jnp.arange(16, dtype=jnp.int32)
            for g in range(CH // 16):
                pg = pgs[g]
                for k2 in range(16):
                    oh = (lanes == k2).astype(jnp.float32)
                    w = jnp.broadcast_to(jnp.sum(pg * oh), (16,))
                    r = g * 16 + k2
                    for q in range(D // 16):
                        rows_v[r, pl.ds(q * 16, 16)] = rows_v[r, pl.ds(q * 16, 16)] * w
            pltpu.sync_copy(rows_v, acc_sh.at[dst_c.at[0]], add=True)
            return carry

        lax.fori_loop(0, CHUNKS, chunk, 0)

        pltpu.sync_copy(den_v, den_hbm.at[wid])
        plsc.subcore_barrier()

        pltpu.sync_copy(acc_sh.at[pl.ds(rbase, RPT)],
                        acc_hbm.at[cid].at[pl.ds(rbase, RPT)])

    return k(z, s1, s2, src, dst, m_splat)


def _tc_c(acc_ref, dp_ref, o_ref):
    d = jnp.sum(dp_ref[:, 0, :], axis=0)
    inv = jnp.where(d > 0, 1.0 / d, 0.0)
    o_ref[...] = (acc_ref[0, :N] + acc_ref[1, :N]) * inv[:, None]


def _combine(acc, den_parts):
    return pl.pallas_call(
        _tc_c,
        out_shape=jax.ShapeDtypeStruct((N, D), jnp.float32),
    )(acc, den_parts)


def kernel(h, edge_index, W_fc, W_attn):
    h_pad = jnp.pad(h, ((1, 1), (0, 0)))
    h_cat = jnp.concatenate([h_pad[:-2], h, h_pad[2:]], axis=1)
    A_pad = jnp.pad(W_attn.reshape(2, D).T, ((0, 0), (0, 126)))

    z, S, m = _dense_front(h_cat, W_fc, A_pad)
    s1 = S[:, 0]
    s2 = S[:, 1]
    M = m[0, 0] + m[0, 1]
    m_splat = jnp.full((16,), M, jnp.float32)

    src = edge_index[0]
    dst = edge_index[1]
    dst2 = dst.reshape(E // CH, CH)
    acc, den_parts = _sc_edges(z, s1, s2, src, dst2, m_splat)
    return _combine(acc, den_parts)


# speedup vs baseline: 16.4835x; 1.0316x over previous
import functools

import jax
import jax.numpy as jnp
from jax import lax
from jax.experimental import pallas as pl
from jax.experimental.pallas import tpu as pltpu
from jax.experimental.pallas import tpu_sc as plsc

N = 10000
E = 320000
D = 128

NC = 2
NS = 16
NW = NC * NS
EPT = E // NW
CH = 48
EPT_P = 10032
CHUNKS = EPT_P // CH
E_P = NW * EPT_P
NP = 10240
RPT = NP // NS

BM = 400


def _tc_a(hc_ref, wfc_ref, ap_ref, z_ref, s_ref, m_ref):
    zb = jnp.dot(hc_ref[...], wfc_ref[...], preferred_element_type=jnp.float32)
    z_ref[...] = zb
    sb = jnp.dot(zb, ap_ref[...], preferred_element_type=jnp.float32)
    s_ref[...] = sb
    bmax = jnp.broadcast_to(jnp.max(sb, axis=0, keepdims=True), (8, 128))

    @pl.when(pl.program_id(0) == 0)
    def _():
        m_ref[...] = bmax

    @pl.when(pl.program_id(0) > 0)
    def _():
        m_ref[...] = jnp.maximum(m_ref[...], bmax)


def _dense_front(h_cat, W_fc, A_pad):
    return pl.pallas_call(
        _tc_a,
        grid=(N // BM,),
        in_specs=[
            pl.BlockSpec((BM, 3 * D), lambda i: (i, 0)),
            pl.BlockSpec((3 * D, D), lambda i: (0, 0)),
            pl.BlockSpec((D, 128), lambda i: (0, 0)),
        ],
        out_specs=[
            pl.BlockSpec((BM, D), lambda i: (i, 0)),
            pl.BlockSpec((BM, 128), lambda i: (i, 0)),
            pl.BlockSpec((8, 128), lambda i: (0, 0)),
        ],
        out_shape=[
            jax.ShapeDtypeStruct((N, D), jnp.float32),
            jax.ShapeDtypeStruct((N, 128), jnp.float32),
            jax.ShapeDtypeStruct((8, 128), jnp.float32),
        ],
    )(h_cat, W_fc, A_pad)


def _sc_edges(z, s1, s2, src, dst, m_splat):
    mesh = plsc.VectorSubcoreMesh(core_axis_name="c", subcore_axis_name="s")

    @functools.partial(
        pl.kernel,
        out_type=[
            jax.ShapeDtypeStruct((NC, NP, D), jnp.float32),
            jax.ShapeDtypeStruct((NW, 1, NP), jnp.float32),
        ],
        mesh=mesh,
        compiler_params=pltpu.CompilerParams(
            needs_layout_passes=False, use_tc_tiling_on_sc=False),
        scratch_types=[
            pltpu.VMEM((N,), jnp.float32),
            pltpu.VMEM((N,), jnp.float32),
            pltpu.VMEM((1, NP), jnp.float32),
            pltpu.VMEM((2, CH), jnp.int32),
            pltpu.VMEM((2, 1, CH), jnp.int32),
            pltpu.VMEM((2, CH, D), jnp.float32),
            pltpu.VMEM((16,), jnp.float32),
            pltpu.VMEM_SHARED((NP, D), jnp.float32),
            pltpu.SemaphoreType.DMA,
            pltpu.SemaphoreType.DMA,
        ],
    )
    def k(z_hbm, s1_hbm, s2_hbm, src_hbm, dst2_hbm, m_hbm,
          acc_hbm, den_hbm,
          s1_v, s2_v, den_v, src_c, dst_c, rows_v, m_v,
          acc_sh, sem0, sem1):
        cid = lax.axis_index("c")
        sid = lax.axis_index("s")
        wid = sid * NC + cid
        base = wid * EPT_P
        cbase = wid * CHUNKS
        sems = (sem0, sem1)

        pltpu.sync_copy(s1_hbm, s1_v)
        pltpu.sync_copy(s2_hbm, s2_v)
        pltpu.sync_copy(m_hbm, m_v)

        zf = jnp.zeros((16,), jnp.float32)

        def zero_den(i, carry):
            den_v[0, pl.ds(i * 16, 16)] = zf
            return carry

        lax.fori_loop(0, NP // 16, zero_den, 0)

        def zero_rows(r, carry):
            for q in range(D // 16):
                rows_v[0, r, pl.ds(q * 16, 16)] = zf
            return carry

        lax.fori_loop(0, CH, zero_rows, 0)

        rbase = sid * RPT
        for t in range(RPT // CH):
            pltpu.sync_copy(rows_v.at[0], acc_sh.at[pl.ds(rbase + t * CH, CH)])
        pltpu.sync_copy(rows_v.at[0, pl.ds(0, RPT - (RPT // CH) * CH)],
                        acc_sh.at[pl.ds(rbase + (RPT // CH) * CH,
                                        RPT - (RPT // CH) * CH)])

        plsc.subcore_barrier()

        mM = m_v[pl.ds(0, 16)]
        lanes = jnp.arange(16, dtype=jnp.int32)

        def stage(c, b):
            pltpu.sync_copy(src_hbm.at[pl.ds(base + c * CH, CH)], src_c.at[b])
            pltpu.sync_copy(dst2_hbm.at[pl.ds(cbase + c, 1)], dst_c.at[b])
            pltpu.async_copy(z_hbm.at[src_c.at[b]], rows_v.at[b], sems[b])

        def wait_gather(b):
            pltpu.make_async_copy(z_hbm.at[src_c.at[b]], rows_v.at[b],
                                  sems[b]).wait()

        def att_den(b):
            pgs = []
            for g in range(CH // 16):
                si = src_c[b, pl.ds(g * 16, 16)]
                di = dst_c[b, 0, pl.ds(g * 16, 16)]
                g1 = plsc.load_gather(s1_v, [si])
                g2 = plsc.load_gather(s2_v, [di])
                p = jnp.exp(g1 + g2 - mM)
                plsc.addupdate_scatter(den_v.at[0], [di], p)
                pgs.append(p)
            return pgs

        def weight_scatter(b, pgs):
            for g in range(CH // 16):
                pg = pgs[g]
                for k2 in range(16):
                    oh = (lanes == k2).astype(jnp.float32)
                    w = jnp.broadcast_to(jnp.sum(pg * oh), (16,))
                    r = g * 16 + k2
                    for q in range(D // 16):
                        rows_v[b, r, pl.ds(q * 16, 16)] = (
                            rows_v[b, r, pl.ds(q * 16, 16)] * w)
            pltpu.sync_copy(rows_v.at[b], acc_sh.at[dst_c.at[b, 0]], add=True)

        stage(0, 0)

        def pair(g, carry):
            c0 = 2 * g
            stage(c0 + 1, 1)
            pgs0 = att_den(0)
            wait_gather(0)
            weight_scatter(0, pgs0)
            stage(c0 + 2, 0)
            pgs1 = att_den(1)
            wait_gather(1)
            weight_scatter(1, pgs1)
            return carry

        lax.fori_loop(0, (CHUNKS - 1) // 2, pair, 0)

        pgs = att_den(0)
        wait_gather(0)
        weight_scatter(0, pgs)

        pltpu.sync_copy(den_v, den_hbm.at[wid])
        plsc.subcore_barrier()

        pltpu.sync_copy(acc_sh.at[pl.ds(rbase, RPT)],
                        acc_hbm.at[cid].at[pl.ds(rbase, RPT)])

    return k(z, s1, s2, src, dst, m_splat)


def _tc_c(acc_ref, dp_ref, o_ref):
    d = jnp.sum(dp_ref[:, 0, :N], axis=0)
    inv = jnp.where(d > 0, 1.0 / d, 0.0)
    o_ref[...] = (acc_ref[0, :N] + acc_ref[1, :N]) * inv[:, None]


def _combine(acc, den_parts):
    return pl.pallas_call(
        _tc_c,
        out_shape=jax.ShapeDtypeStruct((N, D), jnp.float32),
    )(acc, den_parts)


def kernel(h, edge_index, W_fc, W_attn):
    h_pad = jnp.pad(h, ((1, 1), (0, 0)))
    h_cat = jnp.concatenate([h_pad[:-2], h, h_pad[2:]], axis=1)
    A_pad = jnp.pad(W_attn.reshape(2, D).T, ((0, 0), (0, 126)))

    z, S, m = _dense_front(h_cat, W_fc, A_pad)
    s1 = S[:, 0]
    s2 = S[:, 1]
    M = m[0, 0] + m[0, 1]
    m_splat = jnp.full((16,), M, jnp.float32)

    src_p = jnp.pad(edge_index[0].reshape(NW, EPT),
                    ((0, 0), (0, EPT_P - EPT))).reshape(E_P)
    dst_p = jnp.pad(edge_index[1].reshape(NW, EPT),
                    ((0, 0), (0, EPT_P - EPT)),
                    constant_values=N).reshape(E_P // CH, CH)
    acc, den_parts = _sc_edges(z, s1, s2, src_p, dst_p, m_splat)
    return _combine(acc, den_parts)


# speedup vs baseline: 22.6412x; 1.3736x over previous
import functools

import jax
import jax.numpy as jnp
from jax import lax
from jax.experimental import pallas as pl
from jax.experimental.pallas import tpu as pltpu
from jax.experimental.pallas import tpu_sc as plsc

N = 10000
E = 320000
D = 128

NC = 2
NS = 16
NW = NC * NS
EPT = E // NW
CH = 48
EPT_P = 10032
CHUNKS = EPT_P // CH
E_P = NW * EPT_P
SBC = 19
SBE = SBC * CH
NSB = CHUNKS // SBC
NP = 10240
RPT = NP // NS

BM = 400


def _tc_a(hc_ref, wfc_ref, ap_ref, z_ref, s_ref, m_ref):
    zb = jnp.dot(hc_ref[...], wfc_ref[...], preferred_element_type=jnp.float32)
    z_ref[...] = zb
    sb = jnp.dot(zb, ap_ref[...], preferred_element_type=jnp.float32)
    s_ref[...] = sb
    bmax = jnp.broadcast_to(jnp.max(sb, axis=0, keepdims=True), (8, 128))

    @pl.when(pl.program_id(0) == 0)
    def _():
        m_ref[...] = bmax

    @pl.when(pl.program_id(0) > 0)
    def _():
        m_ref[...] = jnp.maximum(m_ref[...], bmax)


def _dense_front(h_cat, W_fc, A_pad):
    return pl.pallas_call(
        _tc_a,
        grid=(N // BM,),
        in_specs=[
            pl.BlockSpec((BM, 3 * D), lambda i: (i, 0)),
            pl.BlockSpec((3 * D, D), lambda i: (0, 0)),
            pl.BlockSpec((D, 128), lambda i: (0, 0)),
        ],
        out_specs=[
            pl.BlockSpec((BM, D), lambda i: (i, 0)),
            pl.BlockSpec((BM, 128), lambda i: (i, 0)),
            pl.BlockSpec((8, 128), lambda i: (0, 0)),
        ],
        out_shape=[
            jax.ShapeDtypeStruct((N, D), jnp.float32),
            jax.ShapeDtypeStruct((N, 128), jnp.float32),
            jax.ShapeDtypeStruct((8, 128), jnp.float32),
        ],
    )(h_cat, W_fc, A_pad)


def _sc_edges(z, s1, s2, src, dst, m_splat):
    mesh = plsc.VectorSubcoreMesh(core_axis_name="c", subcore_axis_name="s")

    @functools.partial(
        pl.kernel,
        out_type=[
            jax.ShapeDtypeStruct((NC, NP, D), jnp.float32),
            jax.ShapeDtypeStruct((NW, 1, NP), jnp.float32),
        ],
        mesh=mesh,
        compiler_params=pltpu.CompilerParams(
            needs_layout_passes=False, use_tc_tiling_on_sc=False),
        scratch_types=[
            pltpu.VMEM((N,), jnp.float32),
            pltpu.VMEM((N,), jnp.float32),
            pltpu.VMEM((1, NP), jnp.float32),
            pltpu.VMEM((SBE,), jnp.int32),
            pltpu.VMEM((SBC, CH), jnp.int32),
            pltpu.VMEM((2, CH, D), jnp.float32),
            pltpu.VMEM((16,), jnp.float32),
            pltpu.VMEM_SHARED((NP, D), jnp.float32),
            pltpu.SemaphoreType.DMA,
            pltpu.SemaphoreType.DMA,
        ],
    )
    def k(z_hbm, s1_hbm, s2_hbm, src_hbm, dst2_hbm, m_hbm,
          acc_hbm, den_hbm,
          s1_v, s2_v, den_v, src_b, dst_b2, rows_v, m_v,
          acc_sh, sem0, sem1):
        cid = lax.axis_index("c")
        sid = lax.axis_index("s")
        wid = sid * NC + cid
        base = wid * EPT_P
        cbase = wid * CHUNKS
        sems = (sem0, sem1)

        pltpu.sync_copy(s1_hbm, s1_v)
        pltpu.sync_copy(s2_hbm, s2_v)
        pltpu.sync_copy(m_hbm, m_v)

        zf = jnp.zeros((16,), jnp.float32)

        def zero_den(i, carry):
            den_v[0, pl.ds(i * 16, 16)] = zf
            return carry

        lax.fori_loop(0, NP // 16, zero_den, 0)

        def zero_rows(r, carry):
            for q in range(D // 16):
                rows_v[0, r, pl.ds(q * 16, 16)] = zf
            return carry

        lax.fori_loop(0, CH, zero_rows, 0)

        rbase = sid * RPT
        for t in range(RPT // CH):
            pltpu.sync_copy(rows_v.at[0], acc_sh.at[pl.ds(rbase + t * CH, CH)])
        pltpu.sync_copy(rows_v.at[0, pl.ds(0, RPT - (RPT // CH) * CH)],
                        acc_sh.at[pl.ds(rbase + (RPT // CH) * CH,
                                        RPT - (RPT // CH) * CH)])

        plsc.subcore_barrier()

        mM = m_v[pl.ds(0, 16)]
        lanes = jnp.arange(16, dtype=jnp.int32)

        def stage_block(sb):
            pltpu.sync_copy(src_hbm.at[pl.ds(base + sb * SBE, SBE)], src_b)
            pltpu.sync_copy(dst2_hbm.at[pl.ds(cbase + sb * SBC, SBC)], dst_b2)

        def gather(j, b):
            pltpu.async_copy(z_hbm.at[src_b.at[pl.ds(j * CH, CH)]],
                             rows_v.at[b], sems[b])

        def wait_gather(j, b):
            pltpu.make_async_copy(z_hbm.at[src_b.at[pl.ds(j * CH, CH)]],
                                  rows_v.at[b], sems[b]).wait()

        def att_den(j):
            pgs = []
            for g in range(CH // 16):
                si = src_b[pl.ds(j * CH + g * 16, 16)]
                di = dst_b2[j, pl.ds(g * 16, 16)]
                g1 = plsc.load_gather(s1_v, [si])
                g2 = plsc.load_gather(s2_v, [di])
                p = jnp.exp(g1 + g2 - mM)
                plsc.addupdate_scatter(den_v.at[0], [di], p)
                pgs.append(p)
            return pgs

        def weight_scatter(b, pgs, j):
            for g in range(CH // 16):
                pg = pgs[g]
                for k2 in range(16):
                    oh = (lanes == k2).astype(jnp.float32)
                    w = jnp.broadcast_to(jnp.sum(pg * oh), (16,))
                    r = g * 16 + k2
                    for q in range(D // 16):
                        rows_v[b, r, pl.ds(q * 16, 16)] = (
                            rows_v[b, r, pl.ds(q * 16, 16)] * w)
            pltpu.sync_copy(rows_v.at[b], acc_sh.at[dst_b2.at[j]], add=True)

        stage_block(0)
        gather(0, 0)

        def block(sb, carry):
            def pairg(g, carry2):
                gather(2 * g + 1, 1)
                pgs0 = att_den(2 * g)
                wait_gather(2 * g, 0)
                weight_scatter(0, pgs0, 2 * g)
                gather(2 * g + 2, 0)
                pgs1 = att_den(2 * g + 1)
                wait_gather(2 * g + 1, 1)
                weight_scatter(1, pgs1, 2 * g + 1)
                return carry2

            lax.fori_loop(0, (SBC - 1) // 2, pairg, 0)
            pgs = att_den(SBC - 1)
            wait_gather(SBC - 1, 0)
            weight_scatter(0, pgs, SBC - 1)

            @pl.when(sb + 1 < NSB)
            def _():
                stage_block(sb + 1)
                gather(0, 0)

            return carry

        lax.fori_loop(0, NSB, block, 0)

        pltpu.sync_copy(den_v, den_hbm.at[wid])
        plsc.subcore_barrier()

        pltpu.sync_copy(acc_sh.at[pl.ds(rbase, RPT)],
                        acc_hbm.at[cid].at[pl.ds(rbase, RPT)])

    return k(z, s1, s2, src, dst, m_splat)


def _tc_c(acc_ref, dp_ref, o_ref):
    d = jnp.sum(dp_ref[:, 0, :N], axis=0)
    inv = jnp.where(d > 0, 1.0 / d, 0.0)
    o_ref[...] = (acc_ref[0, :N] + acc_ref[1, :N]) * inv[:, None]


def _combine(acc, den_parts):
    return pl.pallas_call(
        _tc_c,
        out_shape=jax.ShapeDtypeStruct((N, D), jnp.float32),
    )(acc, den_parts)


def kernel(h, edge_index, W_fc, W_attn):
    h_pad = jnp.pad(h, ((1, 1), (0, 0)))
    h_cat = jnp.concatenate([h_pad[:-2], h, h_pad[2:]], axis=1)
    A_pad = jnp.pad(W_attn.reshape(2, D).T, ((0, 0), (0, 126)))

    z, S, m = _dense_front(h_cat, W_fc, A_pad)
    s1 = S[:, 0]
    s2 = S[:, 1]
    M = m[0, 0] + m[0, 1]
    m_splat = jnp.full((16,), M, jnp.float32)

    src_p = jnp.pad(edge_index[0].reshape(NW, EPT),
                    ((0, 0), (0, EPT_P - EPT))).reshape(E_P)
    dst_p = jnp.pad(edge_index[1].reshape(NW, EPT),
                    ((0, 0), (0, EPT_P - EPT)),
                    constant_values=N).reshape(E_P // CH, CH)
    acc, den_parts = _sc_edges(z, s1, s2, src_p, dst_p, m_splat)
    return _combine(acc, den_parts)


# speedup vs baseline: 22.7500x; 1.0048x over previous
import functools

import jax
import jax.numpy as jnp
from jax import lax
from jax.experimental import pallas as pl
from jax.experimental.pallas import tpu as pltpu
from jax.experimental.pallas import tpu_sc as plsc

N = 10000
E = 320000
D = 128

NC = 2
NS = 16
NW = NC * NS
EPT = E // NW
CH = 48
EPT_P = 10032
CHUNKS = EPT_P // CH
E_P = NW * EPT_P
SBC = 19
SBE = SBC * CH
NSB = CHUNKS // SBC
NP = 10240
RPT = NP // NS

BM = 400


def _tc_a(hc_ref, wfc_ref, ap_ref, z_ref, s_ref, m_ref):
    zb = jnp.dot(hc_ref[...], wfc_ref[...], preferred_element_type=jnp.float32)
    z_ref[...] = zb
    sb = jnp.dot(zb, ap_ref[...], preferred_element_type=jnp.float32)
    s_ref[...] = sb
    bmax = jnp.broadcast_to(jnp.max(sb, axis=0, keepdims=True), (8, 128))

    @pl.when(pl.program_id(0) == 0)
    def _():
        m_ref[...] = bmax

    @pl.when(pl.program_id(0) > 0)
    def _():
        m_ref[...] = jnp.maximum(m_ref[...], bmax)


def _dense_front(h_cat, W_fc, A_pad):
    return pl.pallas_call(
        _tc_a,
        grid=(N // BM,),
        in_specs=[
            pl.BlockSpec((BM, 3 * D), lambda i: (i, 0)),
            pl.BlockSpec((3 * D, D), lambda i: (0, 0)),
            pl.BlockSpec((D, 128), lambda i: (0, 0)),
        ],
        out_specs=[
            pl.BlockSpec((BM, D), lambda i: (i, 0)),
            pl.BlockSpec((BM, 128), lambda i: (i, 0)),
            pl.BlockSpec((8, 128), lambda i: (0, 0)),
        ],
        out_shape=[
            jax.ShapeDtypeStruct((N, D), jnp.float32),
            jax.ShapeDtypeStruct((N, 128), jnp.float32),
            jax.ShapeDtypeStruct((8, 128), jnp.float32),
        ],
    )(h_cat, W_fc, A_pad)


def _sc_edges(z, s1, s2, src, dst, m_splat):
    mesh = plsc.VectorSubcoreMesh(core_axis_name="c", subcore_axis_name="s")

    @functools.partial(
        pl.kernel,
        out_type=[
            jax.ShapeDtypeStruct((NC, NP, D), jnp.float32),
            jax.ShapeDtypeStruct((NW, 1, NP), jnp.float32),
        ],
        mesh=mesh,
        compiler_params=pltpu.CompilerParams(
            needs_layout_passes=False, use_tc_tiling_on_sc=False),
        scratch_types=[
            pltpu.VMEM((N,), jnp.float32),
            pltpu.VMEM((N,), jnp.float32),
            pltpu.VMEM((1, NP), jnp.float32),
            pltpu.VMEM((SBE,), jnp.int32),
            pltpu.VMEM((SBC, CH), jnp.int32),
            pltpu.VMEM((2, CH, D), jnp.float32),
            pltpu.VMEM((16,), jnp.float32),
            pltpu.VMEM_SHARED((NP, D), jnp.float32),
            pltpu.SemaphoreType.DMA,
            pltpu.SemaphoreType.DMA,
            pltpu.SemaphoreType.DMA,
            pltpu.SemaphoreType.DMA,
        ],
    )
    def k(z_hbm, s1_hbm, s2_hbm, src_hbm, dst2_hbm, m_hbm,
          acc_hbm, den_hbm,
          s1_v, s2_v, den_v, src_b, dst_b2, rows_v, m_v,
          acc_sh, sem0, sem1, ssem0, ssem1):
        cid = lax.axis_index("c")
        sid = lax.axis_index("s")
        wid = sid * NC + cid
        base = wid * EPT_P
        cbase = wid * CHUNKS
        sems = (sem0, sem1)
        ssems = (ssem0, ssem1)

        pltpu.sync_copy(s1_hbm, s1_v)
        pltpu.sync_copy(s2_hbm, s2_v)
        pltpu.sync_copy(m_hbm, m_v)

        zf = jnp.zeros((16,), jnp.float32)

        def zero_den(i, carry):
            den_v[0, pl.ds(i * 16, 16)] = zf
            return carry

        lax.fori_loop(0, NP // 16, zero_den, 0)

        def zero_rows(r, carry):
            for q in range(D // 16):
                rows_v[0, r, pl.ds(q * 16, 16)] = zf
            return carry

        lax.fori_loop(0, CH, zero_rows, 0)

        rbase = sid * RPT
        for t in range(RPT // CH):
            pltpu.sync_copy(rows_v.at[0], acc_sh.at[pl.ds(rbase + t * CH, CH)])
        pltpu.sync_copy(rows_v.at[0, pl.ds(0, RPT - (RPT // CH) * CH)],
                        acc_sh.at[pl.ds(rbase + (RPT // CH) * CH,
                                        RPT - (RPT // CH) * CH)])

        plsc.subcore_barrier()

        mM = m_v[pl.ds(0, 16)]
        lanes = jnp.arange(16, dtype=jnp.int32)

        def stage_block(sb):
            pltpu.sync_copy(src_hbm.at[pl.ds(base + sb * SBE, SBE)], src_b)
            pltpu.sync_copy(dst2_hbm.at[pl.ds(cbase + sb * SBC, SBC)], dst_b2)

        def gather(j, b):
            pltpu.async_copy(z_hbm.at[src_b.at[pl.ds(j * CH, CH)]],
                             rows_v.at[b], sems[b])

        def wait_gather(j, b):
            pltpu.make_async_copy(z_hbm.at[src_b.at[pl.ds(j * CH, CH)]],
                                  rows_v.at[b], sems[b]).wait()

        def att_den(j):
            pgs = []
            for g in range(CH // 16):
                si = src_b[pl.ds(j * CH + g * 16, 16)]
                di = dst_b2[j, pl.ds(g * 16, 16)]
                g1 = plsc.load_gather(s1_v, [si])
                g2 = plsc.load_gather(s2_v, [di])
                p = jnp.exp(g1 + g2 - mM)
                plsc.addupdate_scatter(den_v.at[0], [di], p)
                pgs.append(p)
            return pgs

        def weight_scatter(b, pgs, j):
            for g in range(CH // 16):
                pg = pgs[g]
                for k2 in range(16):
                    oh = (lanes == k2).astype(jnp.float32)
                    w = jnp.broadcast_to(jnp.sum(pg * oh), (16,))
                    r = g * 16 + k2
                    for q in range(D // 16):
                        rows_v[b, r, pl.ds(q * 16, 16)] = (
                            rows_v[b, r, pl.ds(q * 16, 16)] * w)
            pltpu.async_copy(rows_v.at[b], acc_sh.at[dst_b2.at[j]], ssems[b],
                             add=True)

        def wait_scatter(b):
            pltpu.make_async_copy(rows_v.at[b], acc_sh.at[dst_b2.at[0]],
                                  ssems[b]).wait()

        stage_block(0)
        gather(0, 0)
        pltpu.async_copy(rows_v.at[0], acc_sh.at[pl.ds(N, CH)], ssem1)

        def block(sb, carry):
            def pairg(g, carry2):
                wait_scatter(1)
                gather(2 * g + 1, 1)
                pgs0 = att_den(2 * g)
                wait_gather(2 * g, 0)
                weight_scatter(0, pgs0, 2 * g)
                pgs1 = att_den(2 * g + 1)
                wait_scatter(0)
                gather(2 * g + 2, 0)
                wait_gather(2 * g + 1, 1)
                weight_scatter(1, pgs1, 2 * g + 1)
                return carry2

            lax.fori_loop(0, (SBC - 1) // 2, pairg, 0)
            pgs = att_den(SBC - 1)
            wait_gather(SBC - 1, 0)
            weight_scatter(0, pgs, SBC - 1)
            wait_scatter(1)
            wait_scatter(0)

            @pl.when(sb + 1 < NSB)
            def _():
                stage_block(sb + 1)
                gather(0, 0)
                pltpu.async_copy(rows_v.at[1], acc_sh.at[pl.ds(N, CH)], ssem1)

            return carry

        lax.fori_loop(0, NSB, block, 0)

        pltpu.sync_copy(den_v, den_hbm.at[wid])
        plsc.subcore_barrier()

        pltpu.sync_copy(acc_sh.at[pl.ds(rbase, RPT)],
                        acc_hbm.at[cid].at[pl.ds(rbase, RPT)])

    return k(z, s1, s2, src, dst, m_splat)


def _tc_c(acc_ref, dp_ref, o_ref):
    d = jnp.sum(dp_ref[:, 0, :N], axis=0)
    inv = jnp.where(d > 0, 1.0 / d, 0.0)
    o_ref[...] = (acc_ref[0, :N] + acc_ref[1, :N]) * inv[:, None]


def _combine(acc, den_parts):
    return pl.pallas_call(
        _tc_c,
        out_shape=jax.ShapeDtypeStruct((N, D), jnp.float32),
    )(acc, den_parts)


def kernel(h, edge_index, W_fc, W_attn):
    h_pad = jnp.pad(h, ((1, 1), (0, 0)))
    h_cat = jnp.concatenate([h_pad[:-2], h, h_pad[2:]], axis=1)
    A_pad = jnp.pad(W_attn.reshape(2, D).T, ((0, 0), (0, 126)))

    z, S, m = _dense_front(h_cat, W_fc, A_pad)
    s1 = S[:, 0]
    s2 = S[:, 1]
    M = m[0, 0] + m[0, 1]
    m_splat = jnp.full((16,), M, jnp.float32)

    src_p = jnp.pad(edge_index[0].reshape(NW, EPT),
                    ((0, 0), (0, EPT_P - EPT))).reshape(E_P)
    dst_p = jnp.pad(edge_index[1].reshape(NW, EPT),
                    ((0, 0), (0, EPT_P - EPT)),
                    constant_values=N).reshape(E_P // CH, CH)
    acc, den_parts = _sc_edges(z, s1, s2, src_p, dst_p, m_splat)
    return _combine(acc, den_parts)
